# baseline (speedup 1.0000x reference)
import jax
import jax.numpy as jnp
from jax.experimental import pallas as pl
from jax.experimental.pallas import tpu as pltpu

_B = 4096
_C = 1000
_G = 8
_R = _B // _G


def _rce_kernel(x_hbm, y_ref, out_ref, buf, sem):
    cp = pltpu.make_async_copy(x_hbm, buf, sem)
    cp.start()

    classes = jax.lax.broadcasted_iota(jnp.int32, (1, _C), 1)
    counts = jnp.zeros((1, _C), jnp.float32)
    for k in range(_G):
        yv = y_ref[k].reshape(_R, 1)
        counts = counts + jnp.sum((yv == classes).astype(jnp.float32),
                                  axis=0, keepdims=True)

    cp.wait()
    colsum = jnp.zeros((1, _C), jnp.float32)
    for k in range(_G):
        colsum = colsum + jnp.sum(buf[pl.ds(k * _R, _R)], axis=0,
                                  keepdims=True)

    s = jnp.sum(colsum * counts, keepdims=True)
    out_ref[...] = 6.0 - (6.0 / (_B * _B)) * s


def kernel(x, y):
    y3 = y.astype(jnp.int32).reshape(_G, 1, _R)
    out = pl.pallas_call(
        _rce_kernel,
        in_specs=[
            pl.BlockSpec(memory_space=pl.ANY),
            pl.BlockSpec((_G, 1, _R), lambda: (0, 0, 0)),
        ],
        out_specs=pl.BlockSpec((1, 1), lambda: (0, 0)),
        out_shape=jax.ShapeDtypeStruct((1, 1), jnp.float32),
        scratch_shapes=[
            pltpu.VMEM((_B, _C), jnp.float32),
            pltpu.SemaphoreType.DMA,
        ],
    )(x, y3)
    return jnp.reshape(out, ())


# speedup vs baseline: 1.0610x; 1.0610x over previous
import jax
import jax.numpy as jnp
from jax import lax
from jax.experimental import pallas as pl
from jax.experimental.pallas import tpu as pltpu

_B = 4096
_C = 1000
_NC = 2
_H = _B // _NC
_G = 4
_R = _H // _G


def _tc_body(x_hbm, y_hbm, cs_hbm, cnt_hbm, buf, ybuf, cs_st, cnt_st,
             sem_x, sem_y, sem_cs, sem_cnt):
    cid = lax.axis_index("core")
    cp_x = pltpu.make_async_copy(x_hbm.at[pl.ds(cid * _H, _H)], buf, sem_x)
    cp_x.start()
    cp_y = pltpu.make_async_copy(y_hbm.at[cid], ybuf, sem_y)
    cp_y.start()
    cp_y.wait()

    classes = jax.lax.broadcasted_iota(jnp.int32, (1, _C), 1)
    yv = ybuf[0].reshape(_H, 1)
    cnt_st[...] = jnp.sum((yv == classes).astype(jnp.float32),
                          axis=0, keepdims=True)
    cp_cnt = pltpu.make_async_copy(cnt_st, cnt_hbm.at[cid], sem_cnt)
    cp_cnt.start()

    cp_x.wait()
    colsum = jnp.zeros((1, _C), jnp.float32)
    for k in range(_G):
        colsum = colsum + jnp.sum(buf[pl.ds(k * _R, _R)], axis=0,
                                  keepdims=True)
    cs_st[...] = colsum
    cp_cs = pltpu.make_async_copy(cs_st, cs_hbm.at[cid], sem_cs)
    cp_cs.start()
    cp_cs.wait()
    cp_cnt.wait()


def _combine_kernel(cs_ref, cnt_ref, out_ref):
    colsum = jnp.sum(cs_ref[...], axis=0)
    counts = jnp.sum(cnt_ref[...], axis=0)
    s = jnp.sum(colsum * counts, keepdims=True)
    out_ref[...] = 6.0 - (6.0 / (_B * _B)) * s


def kernel(x, y):
    y3 = y.astype(jnp.int32).reshape(_NC, 1, _H)
    mesh = pltpu.create_tensorcore_mesh("core")
    k = pl.kernel(
        _tc_body,
        out_type=[
            jax.ShapeDtypeStruct((_NC, 1, _C), jnp.float32),
            jax.ShapeDtypeStruct((_NC, 1, _C), jnp.float32),
        ],
        mesh=mesh,
        scratch_types=[
            pltpu.VMEM((_H, _C), jnp.float32),
            pltpu.VMEM((1, _H), jnp.int32),
            pltpu.VMEM((1, _C), jnp.float32),
            pltpu.VMEM((1, _C), jnp.float32),
            pltpu.SemaphoreType.DMA,
            pltpu.SemaphoreType.DMA,
            pltpu.SemaphoreType.DMA,
            pltpu.SemaphoreType.DMA,
        ],
    )
    cs_p, cnt_p = k(x, y3)
    out = pl.pallas_call(
        _combine_kernel,
        out_shape=jax.ShapeDtypeStruct((1, 1), jnp.float32),
    )(cs_p, cnt_p)
    return jnp.reshape(out, ())
